# baseline (speedup 1.0000x reference)
import functools

import jax
import jax.numpy as jnp
from jax import lax
from jax.experimental import pallas as pl
from jax.experimental.pallas import tpu as pltpu
from jax.experimental.pallas import tpu_sc as plsc

_N = 10000
_E = 320000
_EP = 331776
_D = 128
_HD = 64
_NS = 16
_C = 128
_NCH = _EP // _NS // _C
_RPT = _N // _NS
_RC = 125
_NRC = _RPT // _RC
_K = 10
_ALPHA = 0.1

_mesh = plsc.VectorSubcoreMesh(core_axis_name="c", subcore_axis_name="s")
_params = pltpu.CompilerParams(use_tc_tiling_on_sc=False,
                               needs_layout_passes=False)


def _mul_chunk(gbuf, ebuf, q, wscale):
    @plsc.parallel_loop(0, _C // 16, unroll=2)
    def mul_g(g):
        e0 = g * 16
        for i in range(16):
            qq = 2 * g + i // 8
            wv = plsc.bitcast(ebuf[q, 2 + qq, pl.ds((i % 8) * 16, 16)],
                              jnp.float32) * wscale
            for d in range(_HD // 16):
                sl = pl.ds(d * 16, 16)
                gbuf[e0 + i, sl] = gbuf[e0 + i, sl] * wv


def _edge_pass(xs, ys, epk_hbm, dummy_hbm, ebufs, gbufs, ses, sgs, sss,
               wscale, sid):
    ch0 = sid * _NCH

    def epair(p):
        return epk_hbm.at[pl.ds(ch0 + 2 * p, 2)]

    for k in range(2):
        pltpu.async_copy(epair(k), ebufs[k], ses[k])
    pltpu.make_async_copy(epair(0), ebufs[0], ses[0]).wait()
    pltpu.async_copy(xs.at[ebufs[0].at[0, 0]], gbufs[0], sgs[0])

    def body(i, carry):
        for j in range(6):
            ch = i * 6 + j
            g3 = j % 3
            q = j % 2
            sp = (j // 2) % 3

            @pl.when(ch >= 2)
            def _():
                pltpu.make_async_copy(dummy_hbm, gbufs[(g3 + 1) % 3],
                                      sss[(g3 + 1) % 3]).wait()

            if q == 1:
                @pl.when(ch + 4 < _NCH)
                def _():
                    pltpu.async_copy(epair(ch // 2 + 2), ebufs[(sp + 2) % 3],
                                     ses[(sp + 2) % 3])

            @pl.when(ch + 1 < _NCH)
            def _():
                if q == 1:
                    pltpu.make_async_copy(epair(0), ebufs[(sp + 1) % 3],
                                          ses[(sp + 1) % 3]).wait()
                    pltpu.async_copy(xs.at[ebufs[(sp + 1) % 3].at[0, 0]],
                                     gbufs[(g3 + 1) % 3], sgs[(g3 + 1) % 3])
                else:
                    pltpu.async_copy(xs.at[ebufs[sp].at[1, 0]],
                                     gbufs[(g3 + 1) % 3], sgs[(g3 + 1) % 3])

            pltpu.make_async_copy(dummy_hbm, gbufs[g3], sgs[g3]).wait()
            _mul_chunk(gbufs[g3], ebufs[sp], q, wscale)

            pltpu.async_copy(gbufs[g3], ys.at[ebufs[sp].at[q, 1]], sss[g3],
                             add=True)
        return carry

    lax.fori_loop(0, _NCH // 6, body, 0)
    for k in range(2):
        pltpu.make_async_copy(dummy_hbm, gbufs[(_NCH - 2 + k) % 3],
                              sss[(_NCH - 2 + k) % 3]).wait()


def _zero_sbuf(sbuf):
    def zero_row(r, c):
        for d in range(_HD // 16):
            sbuf[r, pl.ds(d * 16, 16)] = jnp.zeros((16,), jnp.float32)
        return c

    lax.fori_loop(0, _RC, zero_row, 0)


def _prop_body(x_hbm, epk_hbm, out_hbm, xs, ys, *scr):
    ebufs, gbufs, ses, sgs, sss = (scr[0:3], scr[3:6], scr[6:9],
                                   scr[9:12], scr[12:15])
    sbuf = gbufs[0].at[pl.ds(0, _RC), :]
    cid = lax.axis_index("c")
    sid = lax.axis_index("s")
    r0 = sid * _RPT
    c0 = cid * _HD
    dummy_hbm = x_hbm.at[pl.ds(0, _C), pl.ds(0, _HD)]

    def load_chunk(j, c):
        rr = r0 + j * _RC
        pltpu.sync_copy(x_hbm.at[pl.ds(rr, _RC), pl.ds(c0, _HD)], sbuf)
        pltpu.sync_copy(sbuf, xs.at[pl.ds(rr, _RC), :])
        return c

    lax.fori_loop(0, _NRC, load_chunk, 0)

    _zero_sbuf(sbuf)

    def zero_chunk(j, c):
        pltpu.sync_copy(sbuf, ys.at[pl.ds(r0 + j * _RC, _RC), :])
        return c

    lax.fori_loop(0, _NRC, zero_chunk, 0)

    plsc.subcore_barrier()
    _edge_pass(xs, ys, epk_hbm, dummy_hbm, ebufs, gbufs, ses, sgs, sss,
               jnp.float32(1.0), sid)
    plsc.subcore_barrier()

    def store_chunk(j, c):
        rr = r0 + j * _RC
        pltpu.sync_copy(ys.at[pl.ds(rr, _RC), :], sbuf)
        pltpu.sync_copy(sbuf, out_hbm.at[pl.ds(rr, _RC), pl.ds(c0, _HD)])
        return c

    lax.fori_loop(0, _NRC, store_chunk, 0)


_EDGE_SCRATCH = (
    [pltpu.VMEM((2, 18, _C), jnp.int32)] * 3 +
    [pltpu.VMEM((_C, _HD), jnp.float32)] * 3 +
    [pltpu.SemaphoreType.DMA] * 9
)

_sc_propagate = functools.partial(
    pl.kernel,
    out_type=jax.ShapeDtypeStruct((_N, _D), jnp.float32),
    mesh=_mesh,
    compiler_params=_params,
    scratch_types=[
        pltpu.VMEM_SHARED((_N, _HD), jnp.float32),
        pltpu.VMEM_SHARED((_N, _HD), jnp.float32),
    ] + _EDGE_SCRATCH,
)(_prop_body)


def _appnp_body(h2_hbm, b2_hbm, epk_hbm, out_hbm, xa, xb, *scr):
    ebufs, gbufs, ses, sgs, sss = (scr[0:3], scr[3:6], scr[7:10],
                                   scr[10:13], scr[13:16])
    sbuf = gbufs[0].at[pl.ds(0, _RC), :]
    bbuf = scr[6]
    cid = lax.axis_index("c")
    sid = lax.axis_index("s")
    r0 = sid * _RPT
    c0 = cid * _HD
    dummy_hbm = h2_hbm.at[pl.ds(0, _C), pl.ds(0, _HD)]

    def load_chunk(j, c):
        rr = r0 + j * _RC
        pltpu.sync_copy(h2_hbm.at[pl.ds(rr, _RC), pl.ds(c0, _HD)], sbuf)
        pltpu.sync_copy(sbuf, xa.at[pl.ds(rr, _RC), :])
        return c

    lax.fori_loop(0, _NRC, load_chunk, 0)

    _zero_sbuf(sbuf)

    def zero_chunk(j, c):
        pltpu.sync_copy(sbuf, xb.at[pl.ds(r0 + j * _RC, _RC), :])
        return c

    lax.fori_loop(0, _NRC, zero_chunk, 0)

    plsc.subcore_barrier()
    _edge_pass(xa, xb, epk_hbm, dummy_hbm, ebufs, gbufs, ses, sgs, sss,
               jnp.float32(1.0), sid)
    plsc.subcore_barrier()

    pltpu.sync_copy(b2_hbm.at[pl.ds(c0, _HD)], bbuf)

    def h0_chunk(j, c):
        rr = r0 + j * _RC
        pltpu.sync_copy(xb.at[pl.ds(rr, _RC), :], sbuf)

        def bias_row(r, c2):
            for d in range(_HD // 16):
                sl = pl.ds(d * 16, 16)
                sbuf[r, sl] = sbuf[r, sl] + bbuf[sl]
            return c2

        lax.fori_loop(0, _RC, bias_row, 0)
        pltpu.sync_copy(sbuf, xa.at[pl.ds(rr, _RC), :])

        def scale_row(r, c2):
            for d in range(_HD // 16):
                sl = pl.ds(d * 16, 16)
                sbuf[r, sl] = sbuf[r, sl] * _ALPHA
            return c2

        lax.fori_loop(0, _RC, scale_row, 0)
        pltpu.sync_copy(sbuf, out_hbm.at[pl.ds(rr, _RC), pl.ds(c0, _HD)])
        return c

    lax.fori_loop(0, _NRC, h0_chunk, 0)

    w9 = jnp.float32(1.0 - _ALPHA)

    def appnp_iter(rb, wb):
        def init_chunk(j, c):
            rr = r0 + j * _RC
            pltpu.sync_copy(out_hbm.at[pl.ds(rr, _RC), pl.ds(c0, _HD)], sbuf)
            pltpu.sync_copy(sbuf, wb.at[pl.ds(rr, _RC), :])
            return c

        lax.fori_loop(0, _NRC, init_chunk, 0)
        plsc.subcore_barrier()
        _edge_pass(rb, wb, epk_hbm, dummy_hbm, ebufs, gbufs, ses, sgs, sss,
                   w9, sid)
        plsc.subcore_barrier()

    def k_body(k, c):
        appnp_iter(xa, xb)
        appnp_iter(xb, xa)
        return c

    lax.fori_loop(0, _K // 2, k_body, 0)

    def store_chunk(j, c):
        rr = r0 + j * _RC
        pltpu.sync_copy(xa.at[pl.ds(rr, _RC), :], sbuf)
        pltpu.sync_copy(sbuf, out_hbm.at[pl.ds(rr, _RC), pl.ds(c0, _HD)])
        return c

    lax.fori_loop(0, _NRC, store_chunk, 0)


_sc_appnp = functools.partial(
    pl.kernel,
    out_type=jax.ShapeDtypeStruct((_N, _D), jnp.float32),
    mesh=_mesh,
    compiler_params=_params,
    scratch_types=[
        pltpu.VMEM_SHARED((_N, _HD), jnp.float32),
        pltpu.VMEM_SHARED((_N, _HD), jnp.float32),
    ] +
    [pltpu.VMEM((2, 18, _C), jnp.int32)] * 3 +
    [pltpu.VMEM((_C, _HD), jnp.float32)] * 3 +
    [pltpu.VMEM((_HD,), jnp.float32)] +
    [pltpu.SemaphoreType.DMA] * 9,
)(_appnp_body)


_BM = 1000


def _linear_body(x_r, w_r, bi_r, bo_r, o_r):
    xx = x_r[...] + bi_r[...]
    o_r[...] = lax.dot_general(
        xx, w_r[...], (((1,), (1,)), ((), ())),
        preferred_element_type=jnp.float32) + bo_r[...]


def _tc_linear(x, W, bin_, bout):
    return pl.pallas_call(
        _linear_body,
        grid=(_N // _BM,),
        in_specs=[
            pl.BlockSpec((_BM, _D), lambda i: (i, 0)),
            pl.BlockSpec((_D, _D), lambda i: (0, 0)),
            pl.BlockSpec((1, _D), lambda i: (0, 0)),
            pl.BlockSpec((1, _D), lambda i: (0, 0)),
        ],
        out_specs=pl.BlockSpec((_BM, _D), lambda i: (i, 0)),
        out_shape=jax.ShapeDtypeStruct((_N, _D), jnp.float32),
    )(x, W, bin_.reshape(1, _D), bout.reshape(1, _D))


def kernel(features, edge_index, edge_weight, W1, b1, W2, b2):
    pad = _EP - _E
    src = jnp.pad(edge_index[0].astype(jnp.int32), (0, pad)).reshape(-1, 1, _C)
    dst = jnp.pad(edge_index[1].astype(jnp.int32), (0, pad)).reshape(-1, 1, _C)
    ewi = jnp.pad(lax.bitcast_convert_type(edge_weight, jnp.int32), (0, pad))
    ewrep = jnp.broadcast_to(ewi[:, None], (_EP, 16)).reshape(-1, 16, _C)
    epk = jnp.concatenate([src, dst, ewrep], axis=1)
    zeros = jnp.zeros((_D,), jnp.float32)

    h1 = _tc_linear(features, W1, zeros, zeros)
    p1 = _sc_propagate(h1, epk)
    h2 = _tc_linear(p1, W2, b1, zeros)
    out = _sc_appnp(h2, b2, epk)

    return (out, 10)


# speedup vs baseline: 1.0674x; 1.0674x over previous
import functools

import jax
import jax.numpy as jnp
from jax import lax
from jax.experimental import pallas as pl
from jax.experimental.pallas import tpu as pltpu
from jax.experimental.pallas import tpu_sc as plsc

_N = 10000
_E = 320000
_EP = 327680
_D = 128
_HD = 64
_NC = 2
_NS = 16
_C = 128
_NCH = _EP // _NS // _C
_RPT = _N // _NS
_RC = 125
_NRC = _RPT // _RC
_K = 10
_ALPHA = 0.1

_mesh = plsc.VectorSubcoreMesh(core_axis_name="c", subcore_axis_name="s")
_params = pltpu.CompilerParams(use_tc_tiling_on_sc=False,
                               needs_layout_passes=False)


def _mul_chunk(gbuf, ebuf, wscale):
    @plsc.parallel_loop(0, _C // 16, unroll=2)
    def mul_g(g):
        e0 = g * 16
        for i in range(16):
            q = 2 * g + i // 8
            wv = plsc.bitcast(ebuf[2 + q, pl.ds((i % 8) * 16, 16)],
                              jnp.float32) * wscale
            for d in range(_HD // 16):
                sl = pl.ds(d * 16, 16)
                gbuf[e0 + i, sl] = gbuf[e0 + i, sl] * wv


def _edge_pass(xs, ys, epk_hbm, dummy_hbm, ebufs, gbufs, ses, sgs, sss,
               wscale, sid):
    ch0 = sid * _NCH

    for k in range(2):
        pltpu.async_copy(epk_hbm.at[ch0 + k], ebufs[k], ses[k])
    pltpu.make_async_copy(epk_hbm.at[ch0], ebufs[0], ses[0]).wait()
    pltpu.async_copy(xs.at[ebufs[0].at[0]], gbufs[0], sgs[0])

    def body(i, carry):
        for j in range(4):
            ch = i * 4 + j

            @pl.when(ch >= 2)
            def _():
                pltpu.make_async_copy(dummy_hbm, gbufs[(j + 2) % 4],
                                      sss[(j + 2) % 4]).wait()

            @pl.when(ch + 2 < _NCH)
            def _():
                pltpu.async_copy(epk_hbm.at[ch0 + ch + 2],
                                 ebufs[(j + 2) % 4], ses[(j + 2) % 4])

            @pl.when(ch + 1 < _NCH)
            def _():
                pltpu.make_async_copy(epk_hbm.at[ch0], ebufs[(j + 1) % 4],
                                      ses[(j + 1) % 4]).wait()
                pltpu.async_copy(xs.at[ebufs[(j + 1) % 4].at[0]],
                                 gbufs[(j + 1) % 4], sgs[(j + 1) % 4])

            pltpu.make_async_copy(dummy_hbm, gbufs[j], sgs[j]).wait()
            _mul_chunk(gbufs[j], ebufs[j], wscale)

            pltpu.async_copy(gbufs[j], ys.at[ebufs[j].at[1]], sss[j],
                             add=True)
        return carry

    lax.fori_loop(0, _NCH // 4, body, 0)
    for k in range(2):
        pltpu.make_async_copy(dummy_hbm, gbufs[(_NCH - 2 + k) % 4],
                              sss[(_NCH - 2 + k) % 4]).wait()


def _zero_sbuf(sbuf):
    def zero_row(r, c):
        for d in range(_HD // 16):
            sbuf[r, pl.ds(d * 16, 16)] = jnp.zeros((16,), jnp.float32)
        return c

    lax.fori_loop(0, _RC, zero_row, 0)


def _prop_body(x_hbm, epk_hbm, out_hbm, xs, ys, *scr):
    ebufs, gbufs, ses, sgs, sss = (scr[0:4], scr[4:8], scr[9:13],
                                   scr[13:17], scr[17:21])
    sbuf = scr[8]
    cid = lax.axis_index("c")
    sid = lax.axis_index("s")
    r0 = sid * _RPT
    c0 = cid * _HD
    dummy_hbm = x_hbm.at[pl.ds(0, _C), pl.ds(0, _HD)]

    def load_chunk(j, c):
        rr = r0 + j * _RC
        pltpu.sync_copy(x_hbm.at[pl.ds(rr, _RC), pl.ds(c0, _HD)], sbuf)
        pltpu.sync_copy(sbuf, xs.at[pl.ds(rr, _RC), :])
        return c

    lax.fori_loop(0, _NRC, load_chunk, 0)

    _zero_sbuf(sbuf)

    def zero_chunk(j, c):
        pltpu.sync_copy(sbuf, ys.at[pl.ds(r0 + j * _RC, _RC), :])
        return c

    lax.fori_loop(0, _NRC, zero_chunk, 0)

    plsc.subcore_barrier()
    _edge_pass(xs, ys, epk_hbm, dummy_hbm, ebufs, gbufs, ses, sgs, sss,
               jnp.float32(1.0), sid)
    plsc.subcore_barrier()

    def store_chunk(j, c):
        rr = r0 + j * _RC
        pltpu.sync_copy(ys.at[pl.ds(rr, _RC), :], sbuf)
        pltpu.sync_copy(sbuf, out_hbm.at[pl.ds(rr, _RC), pl.ds(c0, _HD)])
        return c

    lax.fori_loop(0, _NRC, store_chunk, 0)


_EDGE_SCRATCH = (
    [pltpu.VMEM((18, _C), jnp.int32)] * 4 +
    [pltpu.VMEM((_C, _HD), jnp.float32)] * 4 +
    [pltpu.VMEM((_RC, _HD), jnp.float32)] +
    [pltpu.SemaphoreType.DMA] * 12
)

_sc_propagate = functools.partial(
    pl.kernel,
    out_type=jax.ShapeDtypeStruct((_N, _D), jnp.float32),
    mesh=_mesh,
    compiler_params=_params,
    scratch_types=[
        pltpu.VMEM_SHARED((_N, _HD), jnp.float32),
        pltpu.VMEM_SHARED((_N, _HD), jnp.float32),
    ] + _EDGE_SCRATCH,
)(_prop_body)


def _appnp_body(h2_hbm, b2_hbm, epk_hbm, out_hbm, xa, xb, *scr):
    ebufs, gbufs, ses, sgs, sss = (scr[0:4], scr[4:8], scr[10:14],
                                   scr[14:18], scr[18:22])
    sbuf = scr[8]
    bbuf = scr[9]
    cid = lax.axis_index("c")
    sid = lax.axis_index("s")
    r0 = sid * _RPT
    c0 = cid * _HD
    dummy_hbm = h2_hbm.at[pl.ds(0, _C), pl.ds(0, _HD)]

    pltpu.sync_copy(b2_hbm.at[pl.ds(c0, _HD)], bbuf)

    def load_chunk(j, c):
        rr = r0 + j * _RC
        pltpu.sync_copy(h2_hbm.at[pl.ds(rr, _RC), pl.ds(c0, _HD)], sbuf)
        pltpu.sync_copy(sbuf, xa.at[pl.ds(rr, _RC), :])
        return c

    lax.fori_loop(0, _NRC, load_chunk, 0)

    _zero_sbuf(sbuf)

    def zero_chunk(j, c):
        pltpu.sync_copy(sbuf, xb.at[pl.ds(r0 + j * _RC, _RC), :])
        return c

    lax.fori_loop(0, _NRC, zero_chunk, 0)

    plsc.subcore_barrier()
    _edge_pass(xa, xb, epk_hbm, dummy_hbm, ebufs, gbufs, ses, sgs, sss,
               jnp.float32(1.0), sid)
    plsc.subcore_barrier()

    def h0_chunk(j, c):
        rr = r0 + j * _RC
        pltpu.sync_copy(xb.at[pl.ds(rr, _RC), :], sbuf)

        def bias_row(r, c2):
            for d in range(_HD // 16):
                sl = pl.ds(d * 16, 16)
                sbuf[r, sl] = sbuf[r, sl] + bbuf[sl]
            return c2

        lax.fori_loop(0, _RC, bias_row, 0)
        pltpu.sync_copy(sbuf, xa.at[pl.ds(rr, _RC), :])

        def scale_row(r, c2):
            for d in range(_HD // 16):
                sl = pl.ds(d * 16, 16)
                sbuf[r, sl] = sbuf[r, sl] * _ALPHA
            return c2

        lax.fori_loop(0, _RC, scale_row, 0)
        pltpu.sync_copy(sbuf, out_hbm.at[pl.ds(rr, _RC), pl.ds(c0, _HD)])
        return c

    lax.fori_loop(0, _NRC, h0_chunk, 0)

    w9 = jnp.float32(1.0 - _ALPHA)

    def appnp_iter(rb, wb):
        def init_chunk(j, c):
            rr = r0 + j * _RC
            pltpu.sync_copy(out_hbm.at[pl.ds(rr, _RC), pl.ds(c0, _HD)], sbuf)
            pltpu.sync_copy(sbuf, wb.at[pl.ds(rr, _RC), :])
            return c

        lax.fori_loop(0, _NRC, init_chunk, 0)
        plsc.subcore_barrier()
        _edge_pass(rb, wb, epk_hbm, dummy_hbm, ebufs, gbufs, ses, sgs, sss,
                   w9, sid)
        plsc.subcore_barrier()

    def k_body(k, c):
        appnp_iter(xa, xb)
        appnp_iter(xb, xa)
        return c

    lax.fori_loop(0, _K // 2, k_body, 0)

    def store_chunk(j, c):
        rr = r0 + j * _RC
        pltpu.sync_copy(xa.at[pl.ds(rr, _RC), :], sbuf)
        pltpu.sync_copy(sbuf, out_hbm.at[pl.ds(rr, _RC), pl.ds(c0, _HD)])
        return c

    lax.fori_loop(0, _NRC, store_chunk, 0)


_sc_appnp = functools.partial(
    pl.kernel,
    out_type=jax.ShapeDtypeStruct((_N, _D), jnp.float32),
    mesh=_mesh,
    compiler_params=_params,
    scratch_types=[
        pltpu.VMEM_SHARED((_N, _HD), jnp.float32),
        pltpu.VMEM_SHARED((_N, _HD), jnp.float32),
    ] +
    [pltpu.VMEM((18, _C), jnp.int32)] * 4 +
    [pltpu.VMEM((_C, _HD), jnp.float32)] * 4 +
    [pltpu.VMEM((_RC, _HD), jnp.float32)] +
    [pltpu.VMEM((_HD,), jnp.float32)] +
    [pltpu.SemaphoreType.DMA] * 12,
)(_appnp_body)


_BM = 1000


def _linear_body(x_r, w_r, bi_r, bo_r, o_r):
    xx = x_r[...] + bi_r[...]
    o_r[...] = lax.dot_general(
        xx, w_r[...], (((1,), (1,)), ((), ())),
        preferred_element_type=jnp.float32) + bo_r[...]


def _tc_linear(x, W, bin_, bout):
    return pl.pallas_call(
        _linear_body,
        grid=(_N // _BM,),
        in_specs=[
            pl.BlockSpec((_BM, _D), lambda i: (i, 0)),
            pl.BlockSpec((_D, _D), lambda i: (0, 0)),
            pl.BlockSpec((1, _D), lambda i: (0, 0)),
            pl.BlockSpec((1, _D), lambda i: (0, 0)),
        ],
        out_specs=pl.BlockSpec((_BM, _D), lambda i: (i, 0)),
        out_shape=jax.ShapeDtypeStruct((_N, _D), jnp.float32),
    )(x, W, bin_.reshape(1, _D), bout.reshape(1, _D))


def kernel(features, edge_index, edge_weight, W1, b1, W2, b2):
    pad = _EP - _E
    src = jnp.pad(edge_index[0].astype(jnp.int32), (0, pad)).reshape(-1, 1, _C)
    dst = jnp.pad(edge_index[1].astype(jnp.int32), (0, pad)).reshape(-1, 1, _C)
    ewi = jnp.pad(lax.bitcast_convert_type(edge_weight, jnp.int32), (0, pad))
    ewrep = jnp.broadcast_to(ewi[:, None], (_EP, 16)).reshape(-1, 16, _C)
    epk = jnp.concatenate([src, dst, ewrep], axis=1)
    zeros = jnp.zeros((_D,), jnp.float32)

    h1 = _tc_linear(features, W1, zeros, zeros)
    p1 = _sc_propagate(h1, epk)
    h2 = _tc_linear(p1, W2, b1, zeros)
    out = _sc_appnp(h2, b2, epk)

    return (out, 10)
